# Initial kernel scaffold; baseline (speedup 1.0000x reference)
import jax
import jax.numpy as jnp
from jax.experimental import pallas as pl


def kernel(loc_data, conf_data, iou_data, priors, targets):
    raise NotImplementedError("write your pallas kernel here")


# speedup vs baseline: 45.2626x; 45.2626x over previous
import functools

import jax
import jax.numpy as jnp
from jax import lax
from jax.experimental import pallas as pl
from jax.experimental.pallas import tpu as pltpu

_NUM_CLASSES = 2
_IOU_THRESHOLD = 0.35
_NEGPOS_RATIO = 3
_V0 = 0.1
_V1 = 0.2
_SMOOTH_POINT = 0.2


def _smooth_l1(x, y):
    d = jnp.abs(x - y)
    return jnp.where(d < 1.0, 0.5 * d * d, d - 0.5)


def _body(loc_ref, conf_ref, iou_ref, pri_ref, tgt_ref,
          o0, o1, o2, o3,
          lc_s, acc_s, np_s, n_s,
          *, B, P, LD, T):
    i = pl.program_id(0)

    @pl.when(i == 0)
    def _init():
        acc_s[0] = 0.0
        acc_s[1] = 0.0
        acc_s[2] = 0.0
        acc_s[3] = 0.0
        n_s[0] = 0

    tgt = tgt_ref[0]
    pr_cx = pri_ref[0]
    pr_cy = pri_ref[1]
    pr_w = pri_ref[2]
    pr_h = pri_ref[3]
    pf_x1 = pr_cx - pr_w * 0.5
    pf_y1 = pr_cy - pr_h * 0.5
    pf_x2 = pr_cx + pr_w * 0.5
    pf_y2 = pr_cy + pr_h * 0.5

    a_x1 = tgt[0][:, None]
    a_y1 = tgt[1][:, None]
    a_x2 = tgt[2][:, None]
    a_y2 = tgt[3][:, None]
    iw = jnp.maximum(jnp.minimum(a_x2, pf_x2[None, :]) -
                     jnp.maximum(a_x1, pf_x1[None, :]), 0.0)
    ih = jnp.maximum(jnp.minimum(a_y2, pf_y2[None, :]) -
                     jnp.maximum(a_y1, pf_y1[None, :]), 0.0)
    inter = iw * ih
    area_a = (a_x2 - a_x1) * (a_y2 - a_y1)
    area_b = ((pf_x2 - pf_x1) * (pf_y2 - pf_y1))[None, :]
    ov = inter / jnp.maximum(area_a + area_b - inter, 1e-12)

    iota_p = lax.broadcasted_iota(jnp.int32, (T, P), 1)
    iota_j = lax.broadcasted_iota(jnp.int32, (T, P), 0)

    rowmax = jnp.max(ov, axis=1, keepdims=True)
    bpi = jnp.min(jnp.where(ov == rowmax, iota_p, P), axis=1,
                  keepdims=True)
    colmax = jnp.max(ov, axis=0)
    bti = jnp.min(jnp.where(ov == colmax[None, :], iota_j, T), axis=0)

    forced_m = iota_p == bpi
    forced = jnp.max(forced_m.astype(jnp.int32), axis=0) > 0
    forced_idx = jnp.max(jnp.where(forced_m, iota_j, -1), axis=0)
    fidx = jnp.where(forced, forced_idx, bti)
    btov = jnp.where(forced, 2.0, colmax)

    onehot = (iota_j == fidx[None, :]).astype(jnp.float32)
    g = lax.dot_general(tgt, onehot, (((1,), (0,)), ((), ())),
                        preferred_element_type=jnp.float32)

    label = g[14]
    conf_ti = jnp.where(btov < _IOU_THRESHOLD, 0, label.astype(jnp.int32))
    pos = conf_ti > 0
    posf = pos.astype(jnp.float32)
    npos = jnp.sum(pos.astype(jnp.int32))

    g_cx = ((g[0] + g[2]) * 0.5 - pr_cx) / (_V0 * pr_w)
    g_cy = ((g[1] + g[3]) * 0.5 - pr_cy) / (_V0 * pr_h)
    g_w = jnp.log(jnp.maximum((g[2] - g[0]) / pr_w, 1e-12)) / _V1
    g_h = jnp.log(jnp.maximum((g[3] - g[1]) / pr_h, 1e-12)) / _V1

    loc = loc_ref[0]
    pcx = loc[0] * _V0
    pcy = loc[1] * _V0
    pw = jnp.exp(loc[2] * _V1)
    ph = jnp.exp(loc[3] * _V1)
    tcx = g_cx * _V0
    tcy = g_cy * _V0
    tw = jnp.exp(g_w * _V1)
    th = jnp.exp(g_h * _V1)
    px1, py1, px2, py2 = pcx - pw * 0.5, pcy - ph * 0.5, pcx + pw * 0.5, pcy + ph * 0.5
    tx1, ty1, tx2, ty2 = tcx - tw * 0.5, tcy - th * 0.5, tcx + tw * 0.5, tcy + th * 0.5
    iw2 = jnp.maximum(jnp.minimum(px2, tx2) - jnp.maximum(px1, tx1), 0.0)
    ih2 = jnp.maximum(jnp.minimum(py2, ty2) - jnp.maximum(py1, ty1), 0.0)
    inter2 = iw2 * ih2
    area_p = (px2 - px1) * (py2 - py1)
    area_t = (tx2 - tx1) * (ty2 - ty1)
    iou2 = inter2 / jnp.maximum(area_p + area_t - inter2, 1e-12)
    l = 1.0 - iou2
    el = jnp.where(l < _SMOOTH_POINT, 0.5 * l * l / _SMOOTH_POINT,
                   l - 0.5 * _SMOOTH_POINT)
    loss_bbox = jnp.sum(el * posf)

    lm_sum = jnp.float32(0.0)
    for r in range(5):
        glx = (g[4 + 2 * r] - pr_cx) / (_V0 * pr_w)
        gly = (g[5 + 2 * r] - pr_cy) / (_V0 * pr_h)
        lm_sum = lm_sum + jnp.sum(_smooth_l1(loc[4 + 2 * r], glx) * posf)
        lm_sum = lm_sum + jnp.sum(_smooth_l1(loc[5 + 2 * r], gly) * posf)

    ih_sum = jnp.sum(_smooth_l1(iou_ref[0, 0], btov) * posf)

    c0 = conf_ref[0, 0]
    c1 = conf_ref[0, 1]
    mx = jnp.maximum(c0, c1)
    lse = jnp.log(jnp.exp(c0 - mx) + jnp.exp(c1 - mx)) + mx
    csel = jnp.where(pos, c1, c0)
    posce = jnp.sum((lse - csel) * posf)
    lc = jnp.where(pos, 0.0, lse - c0)
    lc_s[pl.ds(i, 1), :] = lc[None, :]

    acc_s[0] = acc_s[0] + loss_bbox
    acc_s[1] = acc_s[1] + ih_sum
    acc_s[2] = acc_s[2] + lm_sum
    acc_s[3] = acc_s[3] + posce
    np_s[i] = npos
    n_s[0] = n_s[0] + npos

    @pl.when(i == B - 1)
    def _final():
        kvec = jnp.stack(
            [jnp.minimum(_NEGPOS_RATIO * np_s[j], P - 1) for j in range(B)]
        ).reshape(B, 1)
        vals = lc_s[...]
        bits = lax.bitcast_convert_type(vals, jnp.int32)

        def bit_step(t, prefix):
            cand = prefix | (jnp.int32(1) << (30 - t))
            cnt = jnp.sum((bits >= cand).astype(jnp.int32), axis=1,
                          keepdims=True)
            return jnp.where(cnt >= kvec, cand, prefix)

        tbits = lax.fori_loop(0, 31, bit_step, jnp.zeros((B, 1), jnp.int32))
        tval = lax.bitcast_convert_type(tbits, jnp.float32)
        gt = vals > tval
        cnt_gt = jnp.sum(gt.astype(jnp.int32), axis=1, keepdims=True)
        sum_gt = jnp.sum(vals * gt.astype(jnp.float32), axis=1, keepdims=True)
        topk = sum_gt + (kvec - cnt_gt).astype(jnp.float32) * tval
        neg_sum = jnp.sum(topk)

        nf = jnp.maximum(n_s[0].astype(jnp.float32), 1.0)
        o0[0, 0] = acc_s[0] / nf
        o1[0, 0] = acc_s[1] / nf
        o2[0, 0] = acc_s[2] / (nf * ((LD - 4) / 2.0))
        o3[0, 0] = (acc_s[3] + neg_sum) / nf


def kernel(loc_data, conf_data, iou_data, priors, targets):
    B, P, LD = loc_data.shape
    T = targets.shape[1]
    C = conf_data.shape[2]

    loc_t = jnp.transpose(loc_data, (0, 2, 1))
    conf_t = jnp.transpose(conf_data, (0, 2, 1))
    iou_t = jnp.transpose(iou_data, (0, 2, 1))
    pri_t = jnp.transpose(priors, (1, 0))
    tgt_t = jnp.transpose(targets, (0, 2, 1))

    body = functools.partial(_body, B=B, P=P, LD=LD, T=T)
    scalar = jax.ShapeDtypeStruct((1, 1), jnp.float32)
    out = pl.pallas_call(
        body,
        grid=(B,),
        in_specs=[
            pl.BlockSpec((1, LD, P), lambda i: (i, 0, 0)),
            pl.BlockSpec((1, C, P), lambda i: (i, 0, 0)),
            pl.BlockSpec((1, 1, P), lambda i: (i, 0, 0)),
            pl.BlockSpec((4, P), lambda i: (0, 0)),
            pl.BlockSpec((1, targets.shape[2], T), lambda i: (i, 0, 0)),
        ],
        out_specs=[
            pl.BlockSpec((1, 1), lambda i: (0, 0), memory_space=pltpu.SMEM),
            pl.BlockSpec((1, 1), lambda i: (0, 0), memory_space=pltpu.SMEM),
            pl.BlockSpec((1, 1), lambda i: (0, 0), memory_space=pltpu.SMEM),
            pl.BlockSpec((1, 1), lambda i: (0, 0), memory_space=pltpu.SMEM),
        ],
        out_shape=[scalar, scalar, scalar, scalar],
        scratch_shapes=[
            pltpu.VMEM((B, P), jnp.float32),
            pltpu.SMEM((8,), jnp.float32),
            pltpu.SMEM((B,), jnp.int32),
            pltpu.SMEM((1,), jnp.int32),
        ],
    )(loc_t, conf_t, iou_t, pri_t, tgt_t)
    return (out[0][0, 0], out[1][0, 0], out[2][0, 0], out[3][0, 0])


# speedup vs baseline: 49.6231x; 1.0963x over previous
import functools

import jax
import jax.numpy as jnp
from jax import lax
from jax.experimental import pallas as pl
from jax.experimental.pallas import tpu as pltpu

_NUM_CLASSES = 2
_IOU_THRESHOLD = 0.35
_NEGPOS_RATIO = 3
_V0 = 0.1
_V1 = 0.2
_SMOOTH_POINT = 0.2


def _smooth_l1(x, y):
    d = jnp.abs(x - y)
    return jnp.where(d < 1.0, 0.5 * d * d, d - 0.5)


def _body(loc_ref, conf_ref, iou_ref, pri_ref, tgt_ref,
          o0, o1, o2, o3,
          lc_s, acc_s, np_s, n_s,
          *, B, P, LD, T):
    i = pl.program_id(0)

    @pl.when(i == 0)
    def _init():
        acc_s[0] = 0.0
        acc_s[1] = 0.0
        acc_s[2] = 0.0
        acc_s[3] = 0.0
        n_s[0] = 0

    tgt = tgt_ref[0]
    pr_cx = pri_ref[0]
    pr_cy = pri_ref[1]
    pr_w = pri_ref[2]
    pr_h = pri_ref[3]
    pf_x1 = pr_cx - pr_w * 0.5
    pf_y1 = pr_cy - pr_h * 0.5
    pf_x2 = pr_cx + pr_w * 0.5
    pf_y2 = pr_cy + pr_h * 0.5

    a_x1 = tgt[0][:, None]
    a_y1 = tgt[1][:, None]
    a_x2 = tgt[2][:, None]
    a_y2 = tgt[3][:, None]
    iw = jnp.maximum(jnp.minimum(a_x2, pf_x2[None, :]) -
                     jnp.maximum(a_x1, pf_x1[None, :]), 0.0)
    ih = jnp.maximum(jnp.minimum(a_y2, pf_y2[None, :]) -
                     jnp.maximum(a_y1, pf_y1[None, :]), 0.0)
    inter = iw * ih
    area_a = (a_x2 - a_x1) * (a_y2 - a_y1)
    area_b = ((pf_x2 - pf_x1) * (pf_y2 - pf_y1))[None, :]
    ov = inter / jnp.maximum(area_a + area_b - inter, 1e-12)

    iota_p = lax.broadcasted_iota(jnp.int32, (T, P), 1).astype(jnp.float32)
    iota_j = lax.broadcasted_iota(jnp.int32, (T, P), 0).astype(jnp.float32)

    rowmax = jnp.max(ov, axis=1, keepdims=True)
    bpi = jnp.min(jnp.where(ov == rowmax, iota_p, float(P)), axis=1,
                  keepdims=True)
    colmax = jnp.max(ov, axis=0)
    bti = jnp.min(jnp.where(ov == colmax[None, :], iota_j, float(T)),
                  axis=0)

    forced_m = iota_p == bpi
    forced_idx = jnp.max(jnp.where(forced_m, iota_j, -1.0), axis=0)
    forced = forced_idx >= 0.0
    fidx = jnp.where(forced, forced_idx, bti)
    btov = jnp.where(forced, 2.0, colmax)

    onehot = (iota_j == fidx[None, :]).astype(jnp.float32)
    g = lax.dot_general(tgt, onehot, (((1,), (0,)), ((), ())),
                        preferred_element_type=jnp.float32)

    label = g[14]
    conf_ti = jnp.where(btov < _IOU_THRESHOLD, 0, label.astype(jnp.int32))
    pos = conf_ti > 0
    posf = pos.astype(jnp.float32)
    npos = jnp.sum(pos.astype(jnp.int32))

    g_cx = ((g[0] + g[2]) * 0.5 - pr_cx) / (_V0 * pr_w)
    g_cy = ((g[1] + g[3]) * 0.5 - pr_cy) / (_V0 * pr_h)
    g_w = jnp.log(jnp.maximum((g[2] - g[0]) / pr_w, 1e-12)) / _V1
    g_h = jnp.log(jnp.maximum((g[3] - g[1]) / pr_h, 1e-12)) / _V1

    loc = loc_ref[0]
    pcx = loc[0] * _V0
    pcy = loc[1] * _V0
    pw = jnp.exp(loc[2] * _V1)
    ph = jnp.exp(loc[3] * _V1)
    tcx = g_cx * _V0
    tcy = g_cy * _V0
    tw = jnp.exp(g_w * _V1)
    th = jnp.exp(g_h * _V1)
    px1, py1, px2, py2 = pcx - pw * 0.5, pcy - ph * 0.5, pcx + pw * 0.5, pcy + ph * 0.5
    tx1, ty1, tx2, ty2 = tcx - tw * 0.5, tcy - th * 0.5, tcx + tw * 0.5, tcy + th * 0.5
    iw2 = jnp.maximum(jnp.minimum(px2, tx2) - jnp.maximum(px1, tx1), 0.0)
    ih2 = jnp.maximum(jnp.minimum(py2, ty2) - jnp.maximum(py1, ty1), 0.0)
    inter2 = iw2 * ih2
    area_p = (px2 - px1) * (py2 - py1)
    area_t = (tx2 - tx1) * (ty2 - ty1)
    iou2 = inter2 / jnp.maximum(area_p + area_t - inter2, 1e-12)
    l = 1.0 - iou2
    el = jnp.where(l < _SMOOTH_POINT, 0.5 * l * l / _SMOOTH_POINT,
                   l - 0.5 * _SMOOTH_POINT)
    loss_bbox = jnp.sum(el * posf)

    lm_sum = jnp.float32(0.0)
    for r in range(5):
        glx = (g[4 + 2 * r] - pr_cx) / (_V0 * pr_w)
        gly = (g[5 + 2 * r] - pr_cy) / (_V0 * pr_h)
        lm_sum = lm_sum + jnp.sum(_smooth_l1(loc[4 + 2 * r], glx) * posf)
        lm_sum = lm_sum + jnp.sum(_smooth_l1(loc[5 + 2 * r], gly) * posf)

    ih_sum = jnp.sum(_smooth_l1(iou_ref[0, 0], btov) * posf)

    c0 = conf_ref[0, 0]
    c1 = conf_ref[0, 1]
    mx = jnp.maximum(c0, c1)
    lse = jnp.log(jnp.exp(c0 - mx) + jnp.exp(c1 - mx)) + mx
    csel = jnp.where(pos, c1, c0)
    posce = jnp.sum((lse - csel) * posf)
    lc = jnp.where(pos, 0.0, lse - c0)
    lc_s[pl.ds(i, 1), :] = lc[None, :]

    acc_s[0] = acc_s[0] + loss_bbox
    acc_s[1] = acc_s[1] + ih_sum
    acc_s[2] = acc_s[2] + lm_sum
    acc_s[3] = acc_s[3] + posce
    np_s[i] = npos
    n_s[0] = n_s[0] + npos

    @pl.when(i == B - 1)
    def _final():
        kvec = jnp.stack(
            [jnp.minimum(_NEGPOS_RATIO * np_s[j], P - 1) for j in range(B)]
        ).reshape(B, 1)
        vals = lc_s[...]
        bits = lax.bitcast_convert_type(vals, jnp.int32)

        def bit_step(t, prefix):
            cand = prefix | (jnp.int32(1) << (30 - t))
            cnt = jnp.sum((bits >= cand).astype(jnp.int32), axis=1,
                          keepdims=True)
            return jnp.where(cnt >= kvec, cand, prefix)

        tbits = lax.fori_loop(0, 31, bit_step, jnp.zeros((B, 1), jnp.int32))
        tval = lax.bitcast_convert_type(tbits, jnp.float32)
        gt = vals > tval
        cnt_gt = jnp.sum(gt.astype(jnp.int32), axis=1, keepdims=True)
        sum_gt = jnp.sum(vals * gt.astype(jnp.float32), axis=1, keepdims=True)
        topk = sum_gt + (kvec - cnt_gt).astype(jnp.float32) * tval
        neg_sum = jnp.sum(topk)

        nf = jnp.maximum(n_s[0].astype(jnp.float32), 1.0)
        o0[0, 0] = acc_s[0] / nf
        o1[0, 0] = acc_s[1] / nf
        o2[0, 0] = acc_s[2] / (nf * ((LD - 4) / 2.0))
        o3[0, 0] = (acc_s[3] + neg_sum) / nf


def kernel(loc_data, conf_data, iou_data, priors, targets):
    B, P, LD = loc_data.shape
    T = targets.shape[1]
    C = conf_data.shape[2]

    loc_t = jnp.transpose(loc_data, (0, 2, 1))
    conf_t = jnp.transpose(conf_data, (0, 2, 1))
    iou_t = jnp.transpose(iou_data, (0, 2, 1))
    pri_t = jnp.transpose(priors, (1, 0))
    tgt_t = jnp.transpose(targets, (0, 2, 1))

    body = functools.partial(_body, B=B, P=P, LD=LD, T=T)
    scalar = jax.ShapeDtypeStruct((1, 1), jnp.float32)
    out = pl.pallas_call(
        body,
        grid=(B,),
        in_specs=[
            pl.BlockSpec((1, LD, P), lambda i: (i, 0, 0)),
            pl.BlockSpec((1, C, P), lambda i: (i, 0, 0)),
            pl.BlockSpec((1, 1, P), lambda i: (i, 0, 0)),
            pl.BlockSpec((4, P), lambda i: (0, 0)),
            pl.BlockSpec((1, targets.shape[2], T), lambda i: (i, 0, 0)),
        ],
        out_specs=[
            pl.BlockSpec((1, 1), lambda i: (0, 0), memory_space=pltpu.SMEM),
            pl.BlockSpec((1, 1), lambda i: (0, 0), memory_space=pltpu.SMEM),
            pl.BlockSpec((1, 1), lambda i: (0, 0), memory_space=pltpu.SMEM),
            pl.BlockSpec((1, 1), lambda i: (0, 0), memory_space=pltpu.SMEM),
        ],
        out_shape=[scalar, scalar, scalar, scalar],
        scratch_shapes=[
            pltpu.VMEM((B, P), jnp.float32),
            pltpu.SMEM((8,), jnp.float32),
            pltpu.SMEM((B,), jnp.int32),
            pltpu.SMEM((1,), jnp.int32),
        ],
    )(loc_t, conf_t, iou_t, pri_t, tgt_t)
    return (out[0][0, 0], out[1][0, 0], out[2][0, 0], out[3][0, 0])


# speedup vs baseline: 51.4027x; 1.0359x over previous
import functools

import jax
import jax.numpy as jnp
from jax import lax
from jax.experimental import pallas as pl
from jax.experimental.pallas import tpu as pltpu

_NUM_CLASSES = 2
_IOU_THRESHOLD = 0.35
_NEGPOS_RATIO = 3
_V0 = 0.1
_V1 = 0.2
_SMOOTH_POINT = 0.2


def _smooth_l1(x, y):
    d = jnp.abs(x - y)
    return jnp.where(d < 1.0, 0.5 * d * d, d - 0.5)


def _body(loc_ref, conf_ref, iou_ref, pri_ref, tgt_ref,
          o0, o1, o2, o3,
          lc_s, acc_s, np_s, n_s,
          *, B, P, LD, T):
    i = pl.program_id(0)

    @pl.when(i == 0)
    def _init():
        acc_s[0] = 0.0
        acc_s[1] = 0.0
        acc_s[2] = 0.0
        acc_s[3] = 0.0
        n_s[0] = 0

    tgt = tgt_ref[0]
    pr_cx = pri_ref[0]
    pr_cy = pri_ref[1]
    pr_w = pri_ref[2]
    pr_h = pri_ref[3]
    pf_x1 = pr_cx - pr_w * 0.5
    pf_y1 = pr_cy - pr_h * 0.5
    pf_x2 = pr_cx + pr_w * 0.5
    pf_y2 = pr_cy + pr_h * 0.5

    a_x1 = tgt[0][:, None]
    a_y1 = tgt[1][:, None]
    a_x2 = tgt[2][:, None]
    a_y2 = tgt[3][:, None]
    iw = jnp.maximum(jnp.minimum(a_x2, pf_x2[None, :]) -
                     jnp.maximum(a_x1, pf_x1[None, :]), 0.0)
    ih = jnp.maximum(jnp.minimum(a_y2, pf_y2[None, :]) -
                     jnp.maximum(a_y1, pf_y1[None, :]), 0.0)
    inter = iw * ih
    area_a = (a_x2 - a_x1) * (a_y2 - a_y1)
    area_b = ((pf_x2 - pf_x1) * (pf_y2 - pf_y1))[None, :]
    ov = inter / jnp.maximum(area_a + area_b - inter, 1e-12)

    iota_p = lax.broadcasted_iota(jnp.int32, (T, P), 1).astype(jnp.float32)
    iota_j = lax.broadcasted_iota(jnp.int32, (T, P), 0).astype(jnp.float32)

    rowmax = jnp.max(ov, axis=1, keepdims=True)
    bpi = jnp.min(jnp.where(ov == rowmax, iota_p, float(P)), axis=1,
                  keepdims=True)
    colmax = jnp.max(ov, axis=0)
    bti = jnp.min(jnp.where(ov == colmax[None, :], iota_j, float(T)),
                  axis=0)

    forced_m = iota_p == bpi
    forced_idx = jnp.max(jnp.where(forced_m, iota_j, -1.0), axis=0)
    forced = forced_idx >= 0.0
    fidx = jnp.where(forced, forced_idx, bti)
    btov = jnp.where(forced, 2.0, colmax)

    onehot = (iota_j == fidx[None, :]).astype(jnp.float32)
    g = lax.dot_general(tgt, onehot, (((1,), (0,)), ((), ())),
                        preferred_element_type=jnp.float32)

    label = g[14]
    conf_ti = jnp.where(btov < _IOU_THRESHOLD, 0, label.astype(jnp.int32))
    pos = conf_ti > 0
    posf = pos.astype(jnp.float32)
    npos = jnp.sum(pos.astype(jnp.int32))

    g_cx = ((g[0] + g[2]) * 0.5 - pr_cx) / (_V0 * pr_w)
    g_cy = ((g[1] + g[3]) * 0.5 - pr_cy) / (_V0 * pr_h)
    g_w = jnp.log(jnp.maximum((g[2] - g[0]) / pr_w, 1e-12)) / _V1
    g_h = jnp.log(jnp.maximum((g[3] - g[1]) / pr_h, 1e-12)) / _V1

    loc = loc_ref[0]
    pcx = loc[0] * _V0
    pcy = loc[1] * _V0
    pw = jnp.exp(loc[2] * _V1)
    ph = jnp.exp(loc[3] * _V1)
    tcx = g_cx * _V0
    tcy = g_cy * _V0
    tw = jnp.exp(g_w * _V1)
    th = jnp.exp(g_h * _V1)
    px1, py1, px2, py2 = pcx - pw * 0.5, pcy - ph * 0.5, pcx + pw * 0.5, pcy + ph * 0.5
    tx1, ty1, tx2, ty2 = tcx - tw * 0.5, tcy - th * 0.5, tcx + tw * 0.5, tcy + th * 0.5
    iw2 = jnp.maximum(jnp.minimum(px2, tx2) - jnp.maximum(px1, tx1), 0.0)
    ih2 = jnp.maximum(jnp.minimum(py2, ty2) - jnp.maximum(py1, ty1), 0.0)
    inter2 = iw2 * ih2
    area_p = (px2 - px1) * (py2 - py1)
    area_t = (tx2 - tx1) * (ty2 - ty1)
    iou2 = inter2 / jnp.maximum(area_p + area_t - inter2, 1e-12)
    l = 1.0 - iou2
    el = jnp.where(l < _SMOOTH_POINT, 0.5 * l * l / _SMOOTH_POINT,
                   l - 0.5 * _SMOOTH_POINT)
    loss_bbox = jnp.sum(el * posf)

    pr_c10 = jnp.concatenate([pr_cx[None], pr_cy[None]] * 5, axis=0)
    pr_s10 = jnp.concatenate([(_V0 * pr_w)[None], (_V0 * pr_h)[None]] * 5,
                             axis=0)
    glm = (g[4:14] - pr_c10) / pr_s10
    lm_sum = jnp.sum(_smooth_l1(loc[4:14], glm) * posf[None, :])

    ih_sum = jnp.sum(_smooth_l1(iou_ref[0, 0], btov) * posf)

    c0 = conf_ref[0, 0]
    c1 = conf_ref[0, 1]
    mx = jnp.maximum(c0, c1)
    lse = jnp.log(jnp.exp(c0 - mx) + jnp.exp(c1 - mx)) + mx
    csel = jnp.where(pos, c1, c0)
    posce = jnp.sum((lse - csel) * posf)
    lc = jnp.where(pos, 0.0, lse - c0)
    lc_s[pl.ds(i, 1), :] = lc[None, :]

    acc_s[0] = acc_s[0] + loss_bbox
    acc_s[1] = acc_s[1] + ih_sum
    acc_s[2] = acc_s[2] + lm_sum
    acc_s[3] = acc_s[3] + posce
    np_s[i] = npos
    n_s[0] = n_s[0] + npos

    @pl.when(i == B - 1)
    def _final():
        kvec = jnp.stack(
            [jnp.minimum(_NEGPOS_RATIO * np_s[j], P - 1) for j in range(B)]
        ).reshape(B, 1)
        vals = lc_s[...]
        bits = lax.bitcast_convert_type(vals, jnp.int32)

        def bit_step(t, prefix):
            cand = prefix | (jnp.int32(1) << (30 - t))
            cnt = jnp.sum((bits >= cand).astype(jnp.int32), axis=1,
                          keepdims=True)
            return jnp.where(cnt >= kvec, cand, prefix)

        tbits = lax.fori_loop(0, 31, bit_step, jnp.zeros((B, 1), jnp.int32))
        tval = lax.bitcast_convert_type(tbits, jnp.float32)
        gt = vals > tval
        cnt_gt = jnp.sum(gt.astype(jnp.int32), axis=1, keepdims=True)
        sum_gt = jnp.sum(vals * gt.astype(jnp.float32), axis=1, keepdims=True)
        topk = sum_gt + (kvec - cnt_gt).astype(jnp.float32) * tval
        neg_sum = jnp.sum(topk)

        nf = jnp.maximum(n_s[0].astype(jnp.float32), 1.0)
        o0[0, 0] = acc_s[0] / nf
        o1[0, 0] = acc_s[1] / nf
        o2[0, 0] = acc_s[2] / (nf * ((LD - 4) / 2.0))
        o3[0, 0] = (acc_s[3] + neg_sum) / nf


def kernel(loc_data, conf_data, iou_data, priors, targets):
    B, P, LD = loc_data.shape
    T = targets.shape[1]
    C = conf_data.shape[2]

    loc_t = jnp.transpose(loc_data, (0, 2, 1))
    conf_t = jnp.transpose(conf_data, (0, 2, 1))
    iou_t = jnp.transpose(iou_data, (0, 2, 1))
    pri_t = jnp.transpose(priors, (1, 0))
    tgt_t = jnp.transpose(targets, (0, 2, 1))

    body = functools.partial(_body, B=B, P=P, LD=LD, T=T)
    scalar = jax.ShapeDtypeStruct((1, 1), jnp.float32)
    out = pl.pallas_call(
        body,
        grid=(B,),
        in_specs=[
            pl.BlockSpec((1, LD, P), lambda i: (i, 0, 0)),
            pl.BlockSpec((1, C, P), lambda i: (i, 0, 0)),
            pl.BlockSpec((1, 1, P), lambda i: (i, 0, 0)),
            pl.BlockSpec((4, P), lambda i: (0, 0)),
            pl.BlockSpec((1, targets.shape[2], T), lambda i: (i, 0, 0)),
        ],
        out_specs=[
            pl.BlockSpec((1, 1), lambda i: (0, 0), memory_space=pltpu.SMEM),
            pl.BlockSpec((1, 1), lambda i: (0, 0), memory_space=pltpu.SMEM),
            pl.BlockSpec((1, 1), lambda i: (0, 0), memory_space=pltpu.SMEM),
            pl.BlockSpec((1, 1), lambda i: (0, 0), memory_space=pltpu.SMEM),
        ],
        out_shape=[scalar, scalar, scalar, scalar],
        scratch_shapes=[
            pltpu.VMEM((B, P), jnp.float32),
            pltpu.SMEM((8,), jnp.float32),
            pltpu.SMEM((B,), jnp.int32),
            pltpu.SMEM((1,), jnp.int32),
        ],
    )(loc_t, conf_t, iou_t, pri_t, tgt_t)
    return (out[0][0, 0], out[1][0, 0], out[2][0, 0], out[3][0, 0])
